# baseline (speedup 1.0000x reference)
import functools

import jax
import jax.numpy as jnp
from jax import lax
from jax.experimental import pallas as pl
from jax.experimental.pallas import tpu as pltpu
from jax.experimental.pallas import tpu_sc as plsc

N_HEADS = 16
N_TOK = 2048
N_SEL = N_TOK // 2
ROW_TILE = 128
N_ROW_TILES = N_TOK // ROW_TILE
N_BATCH = 4
D_MODEL = 768
J_CHUNK = 512
LANES = 128
SUBS = 8


def _fused_body(*refs):
    (s_refs, (ids_ref, rowp_ref, colp_ref, sel_ref)) = (refs[:16], refs[16:])
    h = pl.program_id(0)
    i = pl.program_id(1)

    def _folds(x):
        rp = x[:, 0:LANES]
        for k in range(1, N_TOK // LANES):
            rp = rp + x[:, k * LANES:(k + 1) * LANES]
        cp = x[0:SUBS, :]
        for k in range(1, ROW_TILE // SUBS):
            cp = cp + x[k * SUBS:(k + 1) * SUBS, :]
        return rp, cp

    folds = [_folds(r[0]) for r in s_refs]

    def _tree(vals):
        while len(vals) > 1:
            vals = [vals[a] + vals[a + 1] for a in range(0, len(vals), 2)]
        return vals[0]

    rp = _tree([f[0] for f in folds])
    cp = _tree([f[1] for f in folds])

    @pl.when(h == 0)
    def _():
        rowp_ref[pl.ds(i * ROW_TILE, ROW_TILE), :] = rp

    @pl.when(h != 0)
    def _():
        rowp_ref[pl.ds(i * ROW_TILE, ROW_TILE), :] = (
            rowp_ref[pl.ds(i * ROW_TILE, ROW_TILE), :] + rp
        )

    first = jnp.logical_and(h == 0, i == 0)

    @pl.when(first)
    def _():
        colp_ref[...] = cp

    @pl.when(jnp.logical_not(first))
    def _():
        colp_ref[...] = colp_ref[...] + cp

    @pl.when(jnp.logical_and(h == 0, i == N_ROW_TILES - 1))
    def _():
        inv = 1.0 / (N_HEADS * N_TOK)
        parts = []
        for i0 in range(0, N_TOK, J_CHUNK):
            tile = rowp_ref[pl.ds(i0, J_CHUNK), :]
            parts.append(jnp.sum(tile, axis=1).reshape(1, J_CHUNK))
        s1 = jnp.concatenate(parts, axis=1) * inv
        s2 = (jnp.sum(colp_ref[...], axis=0) * inv).reshape(1, N_TOK)

        m1 = jnp.sum(s1) * (1.0 / N_TOK)
        m2 = jnp.sum(s2) * (1.0 / N_TOK)
        v1 = jnp.sum((s1 - m1) ** 2)
        v2 = jnp.sum((s2 - m2) ** 2)
        sig_row = jnp.where(v1 > v2, s1, s2)
        sig_col = sig_row.reshape(N_TOK, 1)

        rank = jnp.zeros((1, N_TOK), jnp.float32)
        for j0 in range(0, N_TOK, J_CHUNK):
            sj = lax.slice(sig_col, (j0, 0), (j0 + J_CHUNK, 1))
            rank = rank + jnp.sum((sj > sig_row).astype(jnp.float32),
                                  axis=0, keepdims=True)
        sel_row = (rank < float(N_SEL)).astype(jnp.float32)
        sel_ref[...] = sel_row
        n_hit = jnp.sum(sel_row)

        @pl.when(n_hit != float(N_SEL))
        def _():
            ii = lax.broadcasted_iota(jnp.int32, (J_CHUNK, N_TOK), 1)
            jj0 = lax.broadcasted_iota(jnp.int32, (J_CHUNK, N_TOK), 0)
            rank2 = jnp.zeros((1, N_TOK), jnp.float32)
            for j0 in range(0, N_TOK, J_CHUNK):
                sj = lax.slice(sig_col, (j0, 0), (j0 + J_CHUNK, 1))
                beats = (sj > sig_row) | ((sj == sig_row) & (jj0 + j0 < ii))
                rank2 = rank2 + jnp.sum(beats.astype(jnp.float32), axis=0,
                                        keepdims=True)
            sel_ref[...] = (rank2 < float(N_SEL)).astype(jnp.float32)

        sel_row = sel_ref[...]
        sel_col = sel_row.reshape(N_TOK, 1)

        pos_row = jnp.concatenate(
            [jnp.zeros((1, 1), jnp.float32), sel_row[:, :-1]], axis=1)
        shift = 1
        while shift < N_TOK:
            shifted = jnp.concatenate(
                [jnp.zeros((1, shift), jnp.float32), pos_row[:, :-shift]],
                axis=1)
            pos_row = pos_row + shifted
            shift *= 2
        pos_col = pos_row.reshape(N_TOK, 1)

        pp = lax.broadcasted_iota(jnp.int32, (J_CHUNK, N_SEL), 1
                                  ).astype(jnp.float32)
        ids = jnp.zeros((1, N_SEL), jnp.float32)
        for i0 in range(0, N_TOK, J_CHUNK):
            si = lax.slice(sel_col, (i0, 0), (i0 + J_CHUNK, 1))
            pi = lax.slice(pos_col, (i0, 0), (i0 + J_CHUNK, 1))
            ival = lax.broadcasted_iota(jnp.int32, (J_CHUNK, N_SEL), 0
                                        ).astype(jnp.float32) + i0
            contrib = si * (pi == pp).astype(jnp.float32) * ival
            ids = ids + jnp.sum(contrib, axis=0, keepdims=True)
        ids = ids.astype(jnp.int32)
        for b in range(N_BATCH):
            ids_ref[0, pl.ds(b * N_SEL, N_SEL)] = ids[0] + b * N_TOK


def _select_ids(scores):
    return pl.pallas_call(
        _fused_body,
        grid=(1, N_ROW_TILES),
        in_specs=[
            pl.BlockSpec((1, ROW_TILE, N_TOK),
                         functools.partial(lambda s, h, i: (h + s, i, 0), s))
            for s in range(N_HEADS)
        ],
        out_specs=pl.BlockSpec((1, N_BATCH * N_SEL), lambda h, i: (0, 0)),
        out_shape=jax.ShapeDtypeStruct((1, N_BATCH * N_SEL), jnp.int32),
        scratch_shapes=[
            pltpu.VMEM((N_TOK, LANES), jnp.float32),
            pltpu.VMEM((SUBS, N_TOK), jnp.float32),
            pltpu.VMEM((1, N_TOK), jnp.float32),
        ],
    )(*([scores] * N_HEADS))


_NC, _NS = 2, 16
_B_PER_W = (N_BATCH * N_SEL) // (_NC * _NS)


@functools.cache
def _sc_gather_fn():
    mesh = plsc.VectorSubcoreMesh(core_axis_name="c", subcore_axis_name="s")

    @functools.partial(
        pl.kernel,
        mesh=mesh,
        out_type=jax.ShapeDtypeStruct((N_BATCH * N_SEL, D_MODEL), jnp.float32),
        scratch_types=[
            pltpu.VMEM((_B_PER_W,), jnp.int32),
            pltpu.VMEM((_B_PER_W, D_MODEL), jnp.float32),
            pltpu.SemaphoreType.DMA,
        ],
    )
    def _sc_gather(table_hbm, idx_hbm, out_hbm, idx_v, rows_v, sem):
        wid = lax.axis_index("s") * _NC + lax.axis_index("c")
        base = wid * _B_PER_W
        pltpu.sync_copy(idx_hbm.at[pl.ds(base, _B_PER_W)], idx_v)
        pltpu.async_copy(table_hbm.at[idx_v], rows_v, sem).wait()
        pltpu.sync_copy(rows_v, out_hbm.at[pl.ds(base, _B_PER_W)])

    return _sc_gather


def kernel(tokens, scores):
    ids4 = _select_ids(scores).reshape(N_BATCH * N_SEL)
    table = tokens.reshape(N_BATCH * N_TOK, D_MODEL)
    out = _sc_gather_fn()(table, ids4)
    return out.reshape(N_BATCH, N_SEL, D_MODEL)


# speedup vs baseline: 1.0275x; 1.0275x over previous
import functools

import jax
import jax.numpy as jnp
from jax import lax
from jax.experimental import pallas as pl
from jax.experimental.pallas import tpu as pltpu
from jax.experimental.pallas import tpu_sc as plsc

N_HEADS = 16
N_TOK = 2048
N_SEL = N_TOK // 2
ROW_TILE = 256
N_ROW_TILES = N_TOK // ROW_TILE
N_BATCH = 4
D_MODEL = 768
J_CHUNK = 512
LANES = 128
SUBS = 8


def _fused_body(*refs):
    (s_refs, (ids_ref, rowp_ref, colp_ref, sel_ref)) = (refs[:8], refs[8:])
    h = pl.program_id(0)
    i = pl.program_id(1)

    def _folds(x):
        rp = x[:, 0:LANES]
        for k in range(1, N_TOK // LANES):
            rp = rp + x[:, k * LANES:(k + 1) * LANES]
        cp = x[0:SUBS, :]
        for k in range(1, ROW_TILE // SUBS):
            cp = cp + x[k * SUBS:(k + 1) * SUBS, :]
        return rp, cp

    folds = [_folds(r[0]) for r in s_refs]

    def _tree(vals):
        while len(vals) > 1:
            vals = [vals[a] + vals[a + 1] for a in range(0, len(vals), 2)]
        return vals[0]

    rp = _tree([f[0] for f in folds])
    cp = _tree([f[1] for f in folds])

    @pl.when(h == 0)
    def _():
        rowp_ref[pl.ds(i * ROW_TILE, ROW_TILE), :] = rp

    @pl.when(h != 0)
    def _():
        rowp_ref[pl.ds(i * ROW_TILE, ROW_TILE), :] = (
            rowp_ref[pl.ds(i * ROW_TILE, ROW_TILE), :] + rp
        )

    first = jnp.logical_and(h == 0, i == 0)

    @pl.when(first)
    def _():
        colp_ref[...] = cp

    @pl.when(jnp.logical_not(first))
    def _():
        colp_ref[...] = colp_ref[...] + cp

    @pl.when(jnp.logical_and(h == 0, i == N_ROW_TILES - 1))
    def _():
        inv = 1.0 / (N_HEADS * N_TOK)
        parts = []
        for i0 in range(0, N_TOK, J_CHUNK):
            tile = rowp_ref[pl.ds(i0, J_CHUNK), :]
            parts.append(jnp.sum(tile, axis=1).reshape(1, J_CHUNK))
        s1 = jnp.concatenate(parts, axis=1) * inv
        s2 = (jnp.sum(colp_ref[...], axis=0) * inv).reshape(1, N_TOK)

        m1 = jnp.sum(s1) * (1.0 / N_TOK)
        m2 = jnp.sum(s2) * (1.0 / N_TOK)
        v1 = jnp.sum((s1 - m1) ** 2)
        v2 = jnp.sum((s2 - m2) ** 2)
        sig_row = jnp.where(v1 > v2, s1, s2)
        sig_col = sig_row.reshape(N_TOK, 1)

        rank = jnp.zeros((1, N_TOK), jnp.float32)
        for j0 in range(0, N_TOK, J_CHUNK):
            sj = lax.slice(sig_col, (j0, 0), (j0 + J_CHUNK, 1))
            rank = rank + jnp.sum((sj > sig_row).astype(jnp.float32),
                                  axis=0, keepdims=True)
        sel_row = (rank < float(N_SEL)).astype(jnp.float32)
        sel_ref[...] = sel_row
        n_hit = jnp.sum(sel_row)

        @pl.when(n_hit != float(N_SEL))
        def _():
            ii = lax.broadcasted_iota(jnp.int32, (J_CHUNK, N_TOK), 1)
            jj0 = lax.broadcasted_iota(jnp.int32, (J_CHUNK, N_TOK), 0)
            rank2 = jnp.zeros((1, N_TOK), jnp.float32)
            for j0 in range(0, N_TOK, J_CHUNK):
                sj = lax.slice(sig_col, (j0, 0), (j0 + J_CHUNK, 1))
                beats = (sj > sig_row) | ((sj == sig_row) & (jj0 + j0 < ii))
                rank2 = rank2 + jnp.sum(beats.astype(jnp.float32), axis=0,
                                        keepdims=True)
            sel_ref[...] = (rank2 < float(N_SEL)).astype(jnp.float32)

        sel_row = sel_ref[...]
        sel_col = sel_row.reshape(N_TOK, 1)

        pos_row = jnp.concatenate(
            [jnp.zeros((1, 1), jnp.float32), sel_row[:, :-1]], axis=1)
        shift = 1
        while shift < N_TOK:
            shifted = jnp.concatenate(
                [jnp.zeros((1, shift), jnp.float32), pos_row[:, :-shift]],
                axis=1)
            pos_row = pos_row + shifted
            shift *= 2
        pos_col = pos_row.reshape(N_TOK, 1)

        pp = lax.broadcasted_iota(jnp.int32, (J_CHUNK, N_SEL), 1
                                  ).astype(jnp.float32)
        ids = jnp.zeros((1, N_SEL), jnp.float32)
        for i0 in range(0, N_TOK, J_CHUNK):
            si = lax.slice(sel_col, (i0, 0), (i0 + J_CHUNK, 1))
            pi = lax.slice(pos_col, (i0, 0), (i0 + J_CHUNK, 1))
            ival = lax.broadcasted_iota(jnp.int32, (J_CHUNK, N_SEL), 0
                                        ).astype(jnp.float32) + i0
            contrib = si * (pi == pp).astype(jnp.float32) * ival
            ids = ids + jnp.sum(contrib, axis=0, keepdims=True)
        ids = ids.astype(jnp.int32)
        for b in range(N_BATCH):
            ids_ref[0, pl.ds(b * N_SEL, N_SEL)] = ids[0] + b * N_TOK


def _select_ids(scores):
    return pl.pallas_call(
        _fused_body,
        grid=(N_HEADS // 8, N_ROW_TILES),
        in_specs=[
            pl.BlockSpec((1, ROW_TILE, N_TOK),
                         functools.partial(lambda s, h, i: (h + s * 2, i, 0), s))
            for s in range(8)
        ],
        out_specs=pl.BlockSpec((1, N_BATCH * N_SEL), lambda h, i: (0, 0)),
        out_shape=jax.ShapeDtypeStruct((1, N_BATCH * N_SEL), jnp.int32),
        scratch_shapes=[
            pltpu.VMEM((N_TOK, LANES), jnp.float32),
            pltpu.VMEM((SUBS, N_TOK), jnp.float32),
            pltpu.VMEM((1, N_TOK), jnp.float32),
        ],
    )(*([scores] * 8))


_NC, _NS = 2, 16
_B_PER_W = (N_BATCH * N_SEL) // (_NC * _NS)


@functools.cache
def _sc_gather_fn():
    mesh = plsc.VectorSubcoreMesh(core_axis_name="c", subcore_axis_name="s")

    @functools.partial(
        pl.kernel,
        mesh=mesh,
        out_type=jax.ShapeDtypeStruct((N_BATCH * N_SEL, D_MODEL), jnp.float32),
        scratch_types=[
            pltpu.VMEM((_B_PER_W,), jnp.int32),
            pltpu.VMEM((_B_PER_W, D_MODEL), jnp.float32),
            pltpu.SemaphoreType.DMA,
        ],
    )
    def _sc_gather(table_hbm, idx_hbm, out_hbm, idx_v, rows_v, sem):
        wid = lax.axis_index("s") * _NC + lax.axis_index("c")
        base = wid * _B_PER_W
        pltpu.sync_copy(idx_hbm.at[pl.ds(base, _B_PER_W)], idx_v)
        pltpu.async_copy(table_hbm.at[idx_v], rows_v, sem).wait()
        pltpu.sync_copy(rows_v, out_hbm.at[pl.ds(base, _B_PER_W)])

    return _sc_gather


def kernel(tokens, scores):
    ids4 = _select_ids(scores).reshape(N_BATCH * N_SEL)
    table = tokens.reshape(N_BATCH * N_TOK, D_MODEL)
    out = _sc_gather_fn()(table, ids4)
    return out.reshape(N_BATCH, N_SEL, D_MODEL)
